# baseline (speedup 1.0000x reference)
import functools

import jax
import jax.numpy as jnp
from jax import lax
from jax.experimental import pallas as pl
from jax.experimental.pallas import tpu as pltpu
from jax.experimental.pallas import tpu_sc as plsc

_N_ROWS = 128
_N_COLS = 32768
_N_WORKERS = 32
_ROWS_PER_WORKER = _N_ROWS // _N_WORKERS
_VECS_PER_ROW = _N_COLS // 16
_UNROLL = 4
_BIG = 1 << 30


def _topk3_update(v, t1, t2, t3, x1, x2, x3, n):
    c1 = v > t1
    c2 = v > t2
    c3 = v > t3
    nt1 = jnp.where(c1, v, t1)
    nx1 = jnp.where(c1, n, x1)
    nt2 = jnp.where(c1, t1, jnp.where(c2, v, t2))
    nx2 = jnp.where(c1, x1, jnp.where(c2, n, x2))
    nt3 = jnp.where(c2, t2, jnp.where(c3, v, t3))
    nx3 = jnp.where(c2, x2, jnp.where(c3, n, x3))
    return nt1, nt2, nt3, nx1, nx2, nx3


def _scan_row(buf):
    neg = jnp.full((16,), -jnp.inf, jnp.float32)
    zero = jnp.zeros((16,), jnp.int32)
    init = (neg, neg, neg, zero, zero, zero, zero)

    def body(i, carry):
        t1, t2, t3, x1, x2, x3, n = carry
        base = i * (16 * _UNROLL)
        for u in range(_UNROLL):
            v = buf[pl.ds(base + u * 16, 16)]
            t1, t2, t3, x1, x2, x3 = _topk3_update(v, t1, t2, t3, x1, x2, x3, n)
            n = n + 1
        return (t1, t2, t3, x1, x2, x3, n)

    return lax.fori_loop(0, _VECS_PER_ROW // _UNROLL, body, init)


def _merge_and_store(carry, rv, ri, j):
    t1, t2, t3, x1, x2, x3, _ = carry
    lane = lax.broadcasted_iota(jnp.int32, (16,), 0)
    g1 = x1 * 16 + lane
    g2 = x2 * 16 + lane
    g3 = x3 * 16 + lane
    ms = []
    gs = []
    for _r in range(3):
        m = jnp.max(t1)
        gi = jnp.min(jnp.where(t1 == m, g1, _BIG))
        win = g1 == gi
        ms.append(m)
        gs.append(gi)
        t1 = jnp.where(win, t2, t1)
        g1 = jnp.where(win, g2, g1)
        t2 = jnp.where(win, t3, t2)
        g2 = jnp.where(win, g3, g2)
        t3 = jnp.where(win, -jnp.inf, t3)
    l0 = lane == 0
    l1 = lane == 1
    l2 = lane == 2
    valv = jnp.where(l0, ms[0],
                     jnp.where(l1, ms[1],
                               jnp.where(l2, ms[2], jnp.float32(0.0))))
    idxv = jnp.where(l0, gs[0],
                     jnp.where(l1, gs[1], jnp.where(l2, gs[2], 0)))
    rv[pl.ds(16 * j, 16)] = valv
    ri[pl.ds(16 * j, 16)] = idxv


def _sc_topk_body(scores_hbm, vals_hbm, idx_hbm, buf, rv, ri, sem):
    wid = lax.axis_index("s") * 2 + lax.axis_index("c")
    r0 = wid * _ROWS_PER_WORKER
    for j in range(_ROWS_PER_WORKER):
        pltpu.async_copy(scores_hbm.at[r0 + j], buf, sem).wait()
        carry = _scan_row(buf)
        _merge_and_store(carry, rv, ri, j)
    pltpu.sync_copy(rv, vals_hbm.at[wid])
    pltpu.sync_copy(ri, idx_hbm.at[wid])


def _sc_topk(scores):
    mesh = plsc.VectorSubcoreMesh(core_axis_name="c", subcore_axis_name="s")
    run = functools.partial(
        pl.kernel,
        mesh=mesh,
        out_type=[
            jax.ShapeDtypeStruct((_N_WORKERS, 16 * _ROWS_PER_WORKER), jnp.float32),
            jax.ShapeDtypeStruct((_N_WORKERS, 16 * _ROWS_PER_WORKER), jnp.int32),
        ],
        scratch_types=[
            pltpu.VMEM((_N_COLS,), jnp.float32),
            pltpu.VMEM((16 * _ROWS_PER_WORKER,), jnp.float32),
            pltpu.VMEM((16 * _ROWS_PER_WORKER,), jnp.int32),
            pltpu.SemaphoreType.DMA,
        ],
        compiler_params=pltpu.CompilerParams(
            needs_layout_passes=False, use_tc_tiling_on_sc=False),
    )(_sc_topk_body)
    vals, idx = run(scores)
    return vals.reshape(_N_ROWS, 16), idx.reshape(_N_ROWS, 16)


def _tc_write_kernel(vals_ref, idx_ref, o_ref):
    r, c = o_ref.shape
    iota = lax.broadcasted_iota(jnp.int32, (r, c), 1)
    v1 = vals_ref[:, 0:1]
    v2 = vals_ref[:, 1:2]
    v3 = vals_ref[:, 2:3]
    inv = jnp.float32(1.0) / jnp.maximum(v1 + v2 + v3, jnp.float32(1e-12))
    out = jnp.zeros((r, c), jnp.float32)
    for k in range(3):
        ik = idx_ref[:, k : k + 1]
        vk = vals_ref[:, k : k + 1]
        out = jnp.where(iota == ik, vk * inv, out)
    o_ref[...] = out


def kernel(scores):
    n, c = scores.shape
    vals, idx = _sc_topk(scores)
    rows_per_block = 8
    grid = n // rows_per_block
    return pl.pallas_call(
        _tc_write_kernel,
        grid=(grid,),
        in_specs=[
            pl.BlockSpec((rows_per_block, 16), lambda i: (i, 0)),
            pl.BlockSpec((rows_per_block, 16), lambda i: (i, 0)),
        ],
        out_specs=pl.BlockSpec((rows_per_block, c), lambda i: (i, 0)),
        out_shape=jax.ShapeDtypeStruct((n, c), scores.dtype),
    )(vals, idx)


# speedup vs baseline: 1.1005x; 1.1005x over previous
import functools

import jax
import jax.numpy as jnp
from jax import lax
from jax.experimental import pallas as pl
from jax.experimental.pallas import tpu as pltpu
from jax.experimental.pallas import tpu_sc as plsc

_N_ROWS = 128
_N_COLS = 32768
_N_WORKERS = 32
_GROUP_ROWS = 8
_HALF_COLS = _N_COLS // 2
_CHUNK_COLS = 2048
_N_CHUNKS = _HALF_COLS // _CHUNK_COLS
_VECS_PER_BLOCK = 16
_BLOCK_COLS = 16 * _VECS_PER_BLOCK
_BLOCKS_PER_CHUNK = _CHUNK_COLS // _BLOCK_COLS


def _insert(v, n, t1, t2, t3, x1, x2, x3):
    c1 = v > t1
    c2 = v > t2
    c3 = v > t3
    nt1 = jnp.where(c1, v, t1)
    nx1 = jnp.where(c1, n, x1)
    nt2 = jnp.where(c1, t1, jnp.where(c2, v, t2))
    nx2 = jnp.where(c1, x1, jnp.where(c2, n, x2))
    nt3 = jnp.where(c2, t2, jnp.where(c3, v, t3))
    nx3 = jnp.where(c2, x2, jnp.where(c3, n, x3))
    return nt1, nt2, nt3, nx1, nx2, nx3


def _scan_chunk_row(buf, r, chunk_vec0, state):
    def block(b, carry):
        t1, t2, t3, x1, x2, x3 = carry
        base = b * _BLOCK_COLS
        m = buf[r, pl.ds(base, 16)]
        for u in range(1, _VECS_PER_BLOCK):
            m = jnp.maximum(m, buf[r, pl.ds(base + u * 16, 16)])
        cnt = plsc.all_reduce_population_count(m > t3)
        hit = cnt[0] > 0

        def detail():
            s = (t1, t2, t3, x1, x2, x3)
            for u in range(_VECS_PER_BLOCK):
                v = buf[r, pl.ds(base + u * 16, 16)]
                n = jnp.full((16,), 0, jnp.int32) + (chunk_vec0 + b * _VECS_PER_BLOCK + u)
                s = _insert(v, n, *s)
            return s

        def skip():
            return (t1, t2, t3, x1, x2, x3)

        return lax.cond(hit, detail, skip)

    return lax.fori_loop(0, _BLOCKS_PER_CHUNK, block, state)


def _load_state(stv, sti, r):
    return (stv[r, pl.ds(0, 16)], stv[r, pl.ds(16, 16)], stv[r, pl.ds(32, 16)],
            sti[r, pl.ds(0, 16)], sti[r, pl.ds(16, 16)], sti[r, pl.ds(32, 16)])


def _store_state(stv, sti, r, s):
    for k in range(3):
        stv[r, pl.ds(16 * k, 16)] = s[k]
        sti[r, pl.ds(16 * k, 16)] = s[3 + k]


def _sc_topk_body(scores_hbm, vals_hbm, idx_hbm, buf_a, buf_b, stv, sti,
                  rv, ri, sem_a, sem_b):
    wid = lax.axis_index("s") * 2 + lax.axis_index("c")
    g = wid // 2
    h = wid % 2
    row0 = g * _GROUP_ROWS
    col0 = h * _HALF_COLS

    neg = jnp.full((16,), -jnp.inf, jnp.float32)
    zero = jnp.zeros((16,), jnp.int32)
    for r in range(_GROUP_ROWS):
        _store_state(stv, sti, r, (neg, neg, neg, zero, zero, zero))

    def chunk_src(c):
        start = pl.multiple_of(col0 + c * _CHUNK_COLS, _CHUNK_COLS)
        return scores_hbm.at[pl.ds(row0, _GROUP_ROWS),
                             pl.ds(start, _CHUNK_COLS)]

    def scan_buf(buf, c):
        chunk_vec0 = c * (_CHUNK_COLS // 16)
        for r in range(_GROUP_ROWS):
            s = _load_state(stv, sti, r)
            s = _scan_chunk_row(buf, r, chunk_vec0, s)
            _store_state(stv, sti, r, s)

    last = _N_CHUNKS - 1
    pltpu.async_copy(chunk_src(0), buf_a, sem_a).wait()

    def pair(p, carry):
        c = p * 2
        cp_b = pltpu.async_copy(chunk_src(jnp.minimum(c + 1, last)), buf_b, sem_b)
        scan_buf(buf_a, c)
        cp_b.wait()
        cp_a = pltpu.async_copy(chunk_src(jnp.minimum(c + 2, last)), buf_a, sem_a)
        scan_buf(buf_b, c + 1)
        cp_a.wait()
        return carry

    lax.fori_loop(0, _N_CHUNKS // 2, pair, 0)

    lane = lax.broadcasted_iota(jnp.int32, (16,), 0)
    big = 1 << 30
    for r in range(_GROUP_ROWS):
        t1, t2, t3, x1, x2, x3 = _load_state(stv, sti, r)
        g1 = x1 * 16 + lane + col0
        g2 = x2 * 16 + lane + col0
        g3 = x3 * 16 + lane + col0
        ms = []
        gs = []
        for _round in range(3):
            mx = jnp.max(t1)
            gi = jnp.min(jnp.where(t1 == mx, g1, big))
            win = g1 == gi
            ms.append(mx)
            gs.append(gi)
            t1 = jnp.where(win, t2, t1)
            g1 = jnp.where(win, g2, g1)
            t2 = jnp.where(win, t3, t2)
            g2 = jnp.where(win, g3, g2)
            t3 = jnp.where(win, -jnp.inf, t3)
        l0 = lane == 0
        l1 = lane == 1
        l2 = lane == 2
        valv = jnp.where(l0, ms[0],
                         jnp.where(l1, ms[1],
                                   jnp.where(l2, ms[2], jnp.float32(0.0))))
        idxv = jnp.where(l0, gs[0],
                         jnp.where(l1, gs[1], jnp.where(l2, gs[2], 0)))
        rv[pl.ds(16 * r, 16)] = valv
        ri[pl.ds(16 * r, 16)] = idxv
    pltpu.sync_copy(rv, vals_hbm.at[wid])
    pltpu.sync_copy(ri, idx_hbm.at[wid])


def _sc_topk(scores):
    mesh = plsc.VectorSubcoreMesh(core_axis_name="c", subcore_axis_name="s")
    run = functools.partial(
        pl.kernel,
        mesh=mesh,
        out_type=[
            jax.ShapeDtypeStruct((_N_WORKERS, 16 * _GROUP_ROWS), jnp.float32),
            jax.ShapeDtypeStruct((_N_WORKERS, 16 * _GROUP_ROWS), jnp.int32),
        ],
        scratch_types=[
            pltpu.VMEM((_GROUP_ROWS, _CHUNK_COLS), jnp.float32),
            pltpu.VMEM((_GROUP_ROWS, _CHUNK_COLS), jnp.float32),
            pltpu.VMEM((_GROUP_ROWS, 128), jnp.float32),
            pltpu.VMEM((_GROUP_ROWS, 128), jnp.int32),
            pltpu.VMEM((16 * _GROUP_ROWS,), jnp.float32),
            pltpu.VMEM((16 * _GROUP_ROWS,), jnp.int32),
            pltpu.SemaphoreType.DMA,
            pltpu.SemaphoreType.DMA,
        ],
        compiler_params=pltpu.CompilerParams(
            needs_layout_passes=False, use_tc_tiling_on_sc=True),
    )(_sc_topk_body)
    vals, idx = run(scores)
    vals = vals.reshape(_N_ROWS // _GROUP_ROWS, 2, _GROUP_ROWS, 16)
    idx = idx.reshape(_N_ROWS // _GROUP_ROWS, 2, _GROUP_ROWS, 16)
    va = vals[:, 0].reshape(_N_ROWS, 16)
    vb = vals[:, 1].reshape(_N_ROWS, 16)
    ia = idx[:, 0].reshape(_N_ROWS, 16)
    ib = idx[:, 1].reshape(_N_ROWS, 16)
    return va, ia, vb, ib


def _lex_ge(av, ai, bv, bi):
    return (av > bv) | ((av == bv) & (ai < bi))


def _tc_write_kernel(va_ref, ia_ref, vb_ref, ib_ref, o_ref):
    r, c = o_ref.shape
    a = [(va_ref[:, k:k + 1], ia_ref[:, k:k + 1]) for k in range(3)]
    b = [(vb_ref[:, k:k + 1], ib_ref[:, k:k + 1]) for k in range(3)]

    def sel(cond, x, y):
        return (jnp.where(cond, x[0], y[0]), jnp.where(cond, x[1], y[1]))

    out_vi = []
    ah, am, al = a
    bh, bm, bl = b
    for _k in range(3):
        ge = _lex_ge(ah[0], ah[1], bh[0], bh[1])
        out_vi.append(sel(ge, ah, bh))
        ah, am, al = sel(ge, am, ah), sel(ge, al, am), al
        bh, bm, bl = sel(~ge, bm, bh), sel(~ge, bl, bm), bl

    denom = out_vi[0][0] + out_vi[1][0] + out_vi[2][0]
    inv = jnp.float32(1.0) / jnp.maximum(denom, jnp.float32(1e-12))
    iota = lax.broadcasted_iota(jnp.int32, (r, c), 1)
    out = jnp.zeros((r, c), jnp.float32)
    for k in range(3):
        vk, ik = out_vi[k]
        out = jnp.where(iota == ik, vk * inv, out)
    o_ref[...] = out


def kernel(scores):
    n, c = scores.shape
    va, ia, vb, ib = _sc_topk(scores)
    rows_per_block = _GROUP_ROWS
    grid = n // rows_per_block
    spec16 = pl.BlockSpec((rows_per_block, 16), lambda i: (i, 0))
    return pl.pallas_call(
        _tc_write_kernel,
        grid=(grid,),
        in_specs=[spec16, spec16, spec16, spec16],
        out_specs=pl.BlockSpec((rows_per_block, c), lambda i: (i, 0)),
        out_shape=jax.ShapeDtypeStruct((n, c), scores.dtype),
    )(va, ia, vb, ib)
